# baseline (speedup 1.0000x reference)
import functools

import jax
import jax.numpy as jnp
from jax import lax
from jax.experimental import pallas as pl
from jax.experimental.pallas import tpu as pltpu
from jax.experimental.pallas import tpu_sc as plsc

VOCAB_ = 1000000
EMBED_ = 64
B_ = 4096
T_ = 200
TP1_ = T_ + 1
TPAD_ = 208
NC_ = 2
NS_ = 16
NW_ = NC_ * NS_
BPW_ = B_ // NW_
CT_ = 8
NCH_ = T_ // CT_
LANES_ = 16
NEG_ = EMBED_ // LANES_
NBG_ = BPW_ // LANES_
XPW_ = BPW_ * EMBED_


def _sc_body(emb2_hbm, pht_hbm, rtt_hbm, tp_hbm, pos2_hbm, out_hbm,
             idx_v, rt_v, pidx_v, prow_v, outc_v, pos_v, tp_v,
             semi, semg, semo):
    wid = lax.axis_index("s") * NC_ + lax.axis_index("c")
    b0 = wid * BPW_

    pltpu.sync_copy(pos2_hbm, pos_v)
    pltpu.sync_copy(tp_hbm, tp_v)

    def fire_prefetch(c, p):
        t0 = c * CT_
        pltpu.async_copy(pht_hbm.at[pl.ds(t0, CT_), pl.ds(b0, BPW_)],
                         idx_v.at[p], semi.at[p])
        pltpu.async_copy(rtt_hbm.at[pl.ds(t0, CT_), pl.ds(b0, BPW_)],
                         rt_v.at[p], semi.at[p])

    def prep_chunk(p):
        pltpu.make_async_copy(pht_hbm.at[pl.ds(0, CT_), pl.ds(0, BPW_)],
                              idx_v.at[p], semi.at[p]).wait()
        pltpu.make_async_copy(rtt_hbm.at[pl.ds(0, CT_), pl.ds(0, BPW_)],
                              rt_v.at[p], semi.at[p]).wait()
        for k in range(CT_):
            for bg in range(NBG_):
                sl = pl.ds(bg * LANES_, LANES_)
                pidx_v[k, sl] = lax.shift_right_logical(idx_v[p, k, sl], 1)

    def fire_gather(k, g):
        pltpu.async_copy(emb2_hbm.at[pidx_v.at[k]], prow_v.at[g], semg.at[g])

    def transpose_select_rows(g, k, pos_row, scale_p):
        for bg in range(NBG_):
            if scale_p is not None:
                svec = rt_v[scale_p, k, pl.ds(bg * LANES_, LANES_)]
                ss = [svec[i] for i in range(LANES_)]
            hsrc = (idx_v[scale_p, k, pl.ds(bg * LANES_, LANES_)]
                    if scale_p is not None
                    else tp_v[wid, pl.ds(bg * LANES_, LANES_)])
            hvec = hsrc & 1
            hh = [hvec[i] for i in range(LANES_)]
            for i0 in range(0, LANES_, 4):
                vals = []
                for i in range(i0, i0 + 4):
                    r = bg * LANES_ + i
                    for j in range(NEG_):
                        ve = prow_v[g, r, pl.ds(j * LANES_, LANES_)]
                        vo = prow_v[g, r, pl.ds(EMBED_ + j * LANES_, LANES_)]
                        v = jnp.where(hh[i] == 1, vo, ve)
                        if pos_row is not None:
                            v = v + pos_row[j]
                        vals.append((i, r, j, v))
                for (i, r, j, v) in vals:
                    v2 = v * ss[i] if scale_p is not None else v
                    outc_v[k, pl.ds(r * EMBED_ + j * LANES_, LANES_)] = v2

    def chunk_compute(c, p):
        t0 = c * CT_
        fire_gather(0, 0)

        @pl.loop(0, CT_ // 2)
        def _pair(kk):
            for dk in range(2):
                k = kk * 2 + dk

                @pl.when(k + 1 < CT_)
                def _():
                    fire_gather(k + 1, (dk + 1) % 2)

                g = dk
                pltpu.make_async_copy(emb2_hbm.at[pidx_v.at[0]],
                                      prow_v.at[g], semg.at[g]).wait()
                pos_row = [pos_v[lax.shift_right_logical(t0 + k, 1),
                                 pl.ds(dk * EMBED_ + j * LANES_, LANES_)]
                           for j in range(NEG_)]
                transpose_select_rows(g, k, pos_row, p)

        pltpu.async_copy(
            outc_v, out_hbm.at[pl.ds(t0, CT_), pl.ds(wid * XPW_, XPW_)],
            semo)

    fire_prefetch(0, 0)

    @pl.loop(0, NCH_)
    def _chunk(c):
        p = lax.rem(c, 2)

        @pl.when(c + 1 < NCH_)
        def _():
            fire_prefetch(c + 1, lax.rem(c + 1, 2))

        prep_chunk(p)

        @pl.when(c >= 1)
        def _():
            pltpu.make_async_copy(
                outc_v, out_hbm.at[pl.ds(0, CT_), pl.ds(0, XPW_)],
                semo).wait()

        chunk_compute(c, p)

    for bg in range(NBG_):
        sl = pl.ds(bg * LANES_, LANES_)
        pidx_v[0, sl] = lax.shift_right_logical(tp_v[wid, sl], 1)
    pltpu.make_async_copy(outc_v,
                          out_hbm.at[pl.ds(0, CT_), pl.ds(0, XPW_)],
                          semo).wait()
    pltpu.async_copy(emb2_hbm.at[pidx_v.at[0]], prow_v.at[0], semg.at[0])
    pltpu.make_async_copy(emb2_hbm.at[pidx_v.at[0]], prow_v.at[0],
                          semg.at[0]).wait()
    transpose_select_rows(0, 0, None, None)
    pltpu.sync_copy(outc_v,
                    out_hbm.at[pl.ds(T_, CT_), pl.ds(wid * XPW_, XPW_)])


@jax.jit
def _run_sc(emb2, pht, rtt, tp2, pos2):
    mesh = plsc.VectorSubcoreMesh(core_axis_name="c", subcore_axis_name="s")
    fn = pl.kernel(
        _sc_body,
        out_type=jax.ShapeDtypeStruct((TPAD_, NW_ * XPW_), jnp.float32),
        mesh=mesh,
        scratch_types=[
            pltpu.VMEM((2, CT_, BPW_), jnp.int32),
            pltpu.VMEM((2, CT_, BPW_), jnp.float32),
            pltpu.VMEM((CT_, BPW_), jnp.int32),
            pltpu.VMEM((2, BPW_, 2 * EMBED_), jnp.float32),
            pltpu.VMEM((CT_, XPW_), jnp.float32),
            pltpu.VMEM((T_ // 2, 2 * EMBED_), jnp.float32),
            pltpu.VMEM((NW_, BPW_), jnp.int32),
            pltpu.SemaphoreType.DMA((2,)),
            pltpu.SemaphoreType.DMA((2,)),
            pltpu.SemaphoreType.DMA,
        ],
        compiler_params=pltpu.CompilerParams(use_tc_tiling_on_sc=True),
    )
    return fn(emb2, pht, rtt, tp2, pos2)


def kernel(user_id, product_history, target_product_id,
           product_history_ratings, emb_weights, pos_weights):
    del user_id
    emb2 = emb_weights.reshape(VOCAB_ // 2, 2 * EMBED_)
    pht = product_history.astype(jnp.int32).T
    rtt = product_history_ratings.T
    tp2 = target_product_id.astype(jnp.int32).reshape(NW_, BPW_)
    pos2 = pos_weights.reshape(T_ // 2, 2 * EMBED_)
    out208 = _run_sc(emb2, pht, rtt, tp2, pos2)
    out = out208[:TP1_].reshape(TP1_, B_, EMBED_).transpose(1, 0, 2)
    return out


# speedup vs baseline: 1.1766x; 1.1766x over previous
import functools

import jax
import jax.numpy as jnp
from jax import lax
from jax.experimental import pallas as pl
from jax.experimental.pallas import tpu as pltpu
from jax.experimental.pallas import tpu_sc as plsc

VOCAB_ = 1000000
EMBED_ = 64
B_ = 4096
T_ = 200
TP1_ = T_ + 1
N_ = B_ * TP1_
NC_ = 2
NS_ = 16
NW_ = NC_ * NS_
BPW_ = B_ // NW_
NB_ = 2
CH_ = NB_ * T_
NCH_ = BPW_ // NB_
NBUF_ = 4
LANES_ = 16
GATHER_SPLIT_ = [(0, 128), (128, 128), (256, 128), (384, 16)]


def _sc_body(emb_hbm, ph_hbm, rt_hbm, tp_hbm, pos_hbm, out_hbm,
             idx_v, scale_v, rows_v, pos_v, orow_v,
             semi, semg, semo, semt):
    wid = lax.axis_index("s") * NC_ + lax.axis_index("c")
    b0 = wid * BPW_

    tgt = rows_v.at[0, pl.ds(0, BPW_)]
    pltpu.sync_copy(tp_hbm.at[pl.ds(b0, BPW_)], orow_v)
    pltpu.async_copy(emb_hbm.at[orow_v], tgt, semt)

    pltpu.sync_copy(pos_hbm, pos_v)

    def fire_prefetch(c, p):
        h0 = (b0 + c * NB_) * T_
        pltpu.async_copy(ph_hbm.at[pl.ds(h0, CH_)], idx_v.at[p], semi.at[p])
        pltpu.async_copy(rt_hbm.at[pl.ds(h0, CH_)], scale_v.at[p], semi.at[p])

    def fire_gathers(c, p):
        pltpu.make_async_copy(ph_hbm.at[pl.ds(0, CH_)], idx_v.at[p],
                              semi.at[p]).wait()
        pltpu.make_async_copy(rt_hbm.at[pl.ds(0, CH_)], scale_v.at[p],
                              semi.at[p]).wait()

        if not (isinstance(c, int) and c < NBUF_):
            @pl.when(c >= NBUF_)
            def _():
                for bb in range(NB_):
                    pltpu.make_async_copy(
                        rows_v.at[p, pl.ds(bb * T_, T_)],
                        out_hbm.at[pl.ds(0, T_)], semo.at[p]).wait()

        for off, cnt in GATHER_SPLIT_:
            pltpu.async_copy(emb_hbm.at[idx_v.at[p, pl.ds(off, cnt)]],
                             rows_v.at[p, pl.ds(off, cnt)], semg.at[p])

    def compute_and_write(c, p):
        for off, cnt in GATHER_SPLIT_:
            pltpu.make_async_copy(emb_hbm.at[idx_v.at[p, pl.ds(off, cnt)]],
                                  rows_v.at[p, pl.ds(off, cnt)],
                                  semg.at[p]).wait()

        @pl.loop(0, CH_ // LANES_)
        def _rowgrp(g):
            r0 = g * LANES_
            tvec = lax.rem(r0 + lax.iota(jnp.int32, LANES_), T_)
            svec = scale_v[p, pl.ds(r0, LANES_)]
            ts = [tvec[i] for i in range(LANES_)]
            for i0 in range(0, LANES_, 4):
                vals = []
                for i in range(i0, i0 + 4):
                    r = r0 + i
                    for j in range(EMBED_ // LANES_):
                        sl = pl.ds(j * LANES_, LANES_)
                        vals.append(
                            (i, r, sl, rows_v[p, r, sl] + pos_v[ts[i], sl]))
                for (i, r, sl, v) in vals:
                    rows_v[p, r, sl] = v * svec[i]


        for bb in range(NB_):
            pltpu.async_copy(
                rows_v.at[p, pl.ds(bb * T_, T_)],
                out_hbm.at[pl.ds((b0 + c * NB_ + bb) * TP1_, T_)],
                semo.at[p])

    pltpu.make_async_copy(emb_hbm.at[orow_v], tgt, semt).wait()
    for g in range(BPW_ // LANES_):
        orow_v[pl.ds(g * LANES_, LANES_)] = (
            (b0 + g * LANES_) * TP1_ + T_
            + lax.iota(jnp.int32, LANES_) * TP1_)
    pltpu.async_copy(tgt, out_hbm.at[orow_v], semt).wait()

    fire_prefetch(0, 0)
    fire_prefetch(1, 1)
    fire_gathers(0, 0)

    @pl.loop(0, NCH_)
    def _chunk(c):
        @pl.when(c + 2 < NCH_)
        def _():
            fire_prefetch(c + 2, lax.rem(c + 2, NBUF_))

        @pl.when(c + 1 < NCH_)
        def _():
            fire_gathers(c + 1, lax.rem(c + 1, NBUF_))

        compute_and_write(c, lax.rem(c, NBUF_))

    for p in range(NBUF_):
        for bb in range(NB_):
            pltpu.make_async_copy(
                rows_v.at[p, pl.ds(bb * T_, T_)],
                out_hbm.at[pl.ds(0, T_)], semo.at[p]).wait()


@jax.jit
def _run_sc(emb_weights, ph_flat, rt_flat, tp_flat, pos_weights):
    mesh = plsc.VectorSubcoreMesh(core_axis_name="c", subcore_axis_name="s")
    fn = pl.kernel(
        _sc_body,
        out_type=jax.ShapeDtypeStruct((N_, EMBED_), jnp.float32),
        mesh=mesh,
        scratch_types=[
            pltpu.VMEM((NBUF_, CH_), jnp.int32),
            pltpu.VMEM((NBUF_, CH_), jnp.float32),
            pltpu.VMEM((NBUF_, CH_, EMBED_), jnp.float32),
            pltpu.VMEM((T_, EMBED_), jnp.float32),
            pltpu.VMEM((BPW_,), jnp.int32),
            pltpu.SemaphoreType.DMA((NBUF_,)),
            pltpu.SemaphoreType.DMA((NBUF_,)),
            pltpu.SemaphoreType.DMA((NBUF_,)),
            pltpu.SemaphoreType.DMA,
        ],
        compiler_params=pltpu.CompilerParams(use_tc_tiling_on_sc=False),
    )
    return fn(emb_weights, ph_flat, rt_flat, tp_flat, pos_weights)


def kernel(user_id, product_history, target_product_id,
           product_history_ratings, emb_weights, pos_weights):
    del user_id
    ph_flat = product_history.astype(jnp.int32).reshape(B_ * T_)
    tp_flat = target_product_id.astype(jnp.int32).reshape(B_)
    rt_flat = product_history_ratings.reshape(B_ * T_)
    out = _run_sc(emb_weights, ph_flat, rt_flat, tp_flat, pos_weights)
    return out.reshape(B_, TP1_, EMBED_)
